# baseline (speedup 1.0000x reference)
import functools

import jax
import jax.numpy as jnp
from jax import lax
from jax.experimental import pallas as pl
from jax.experimental.pallas import tpu as pltpu
from jax.experimental.pallas import tpu_sc as plsc

_NC = 2
_NS = 16
_NW = _NC * _NS
_C = 80


def _sc_gather(table, idx3d, n_chunks, out_rows, d):
    mesh = plsc.VectorSubcoreMesh(core_axis_name="c", subcore_axis_name="s")
    total_chunks = out_rows // _C

    assert n_chunks % 3 == 1

    @functools.partial(
        pl.kernel,
        out_type=jax.ShapeDtypeStruct((out_rows, d), table.dtype),
        mesh=mesh,
        scratch_types=[
            pltpu.VMEM((n_chunks, _C), jnp.int32),
            pltpu.VMEM((_C, d), table.dtype),
            pltpu.VMEM((_C, d), table.dtype),
            pltpu.VMEM((_C, d), table.dtype),
            pltpu.SemaphoreType.DMA,
            pltpu.SemaphoreType.DMA,
            pltpu.SemaphoreType.DMA,
            pltpu.SemaphoreType.DMA,
            pltpu.SemaphoreType.DMA,
            pltpu.SemaphoreType.DMA,
        ],
    )
    def k(table_hbm, idx_hbm, out_hbm, idx_v, b0, b1, b2, g0, g1, g2, w0, w1, w2):
        wid = lax.axis_index("s") * _NC + lax.axis_index("c")
        pltpu.sync_copy(idx_hbm.at[wid], idx_v)
        base = wid * n_chunks

        def sg(j, buf, sem):
            pltpu.make_async_copy(table_hbm.at[idx_v.at[j]], buf, sem).start()

        def sw(j, buf, sem):
            off = jnp.minimum(base + j, total_chunks - 1) * _C
            pltpu.make_async_copy(buf, out_hbm.at[pl.ds(off, _C)], sem).start()

        def gwait(buf, sem):
            pltpu.make_async_copy(table_hbm.at[idx_v.at[0]], buf, sem).wait()

        def wwait(buf, sem):
            pltpu.make_async_copy(buf, out_hbm.at[pl.ds(0, _C)], sem).wait()

        last_p = n_chunks // 3 - 1
        sg(0, b0, g0)
        sg(1, b1, g1)

        @pl.loop(0, n_chunks // 3)
        def _(p):
            j0 = 3 * p
            gwait(b0, g0)

            @pl.when(p > 0)
            def _():
                wwait(b2, w2)

            sg(j0 + 2, b2, g2)
            sw(j0, b0, w0)

            gwait(b1, g1)
            wwait(b0, w0)
            sg(j0 + 3, b0, g0)
            sw(j0 + 1, b1, w1)

            gwait(b2, g2)
            wwait(b1, w1)

            @pl.when(p < last_p)
            def _():
                sg(j0 + 4, b1, g1)

            sw(j0 + 2, b2, w2)

        gwait(b0, g0)
        wwait(b2, w2)
        sw(n_chunks - 1, b0, w0)
        wwait(b0, w0)

    return k(table, idx3d)


def kernel(tokenized_prompts, token_embedding_table):
    n, ctx = tokenized_prompts.shape
    _, d = token_embedding_table.shape
    b = n * ctx
    sweep = _C * _NW
    b_pad = ((b + sweep - 1) // sweep) * sweep
    n_chunks = b_pad // sweep
    if n_chunks % 3 != 1:
        n_chunks += (1 - n_chunks % 3) % 3
        b_pad = n_chunks * sweep
    flat = tokenized_prompts.T.reshape(-1)
    pad_chunks = (b_pad - b) // _C
    if pad_chunks:
        flat = jnp.concatenate([flat] + [flat[b - _C:]] * pad_chunks)
    idx3d = flat.reshape(_NW, n_chunks, _C)
    out = _sc_gather(token_embedding_table, idx3d, n_chunks, b, d)
    prompts = out.reshape(ctx, n, d).transpose(1, 0, 2)
    return (prompts, tokenized_prompts)


# speedup vs baseline: 1.0072x; 1.0072x over previous
import functools

import jax
import jax.numpy as jnp
from jax import lax
from jax.experimental import pallas as pl
from jax.experimental.pallas import tpu as pltpu
from jax.experimental.pallas import tpu_sc as plsc

_NC = 2
_NS = 16
_NW = _NC * _NS
_C = 56


def _sc_gather(table, idx3d, n_chunks, out_rows, d):
    mesh = plsc.VectorSubcoreMesh(core_axis_name="c", subcore_axis_name="s")
    total_chunks = out_rows // _C

    assert n_chunks % 4 == 2

    @functools.partial(
        pl.kernel,
        out_type=jax.ShapeDtypeStruct((out_rows, d), table.dtype),
        mesh=mesh,
        scratch_types=[
            pltpu.VMEM((n_chunks, _C), jnp.int32),
            pltpu.VMEM((_C, d), table.dtype),
            pltpu.VMEM((_C, d), table.dtype),
            pltpu.VMEM((_C, d), table.dtype),
            pltpu.VMEM((_C, d), table.dtype),
            pltpu.SemaphoreType.DMA,
            pltpu.SemaphoreType.DMA,
            pltpu.SemaphoreType.DMA,
            pltpu.SemaphoreType.DMA,
            pltpu.SemaphoreType.DMA,
            pltpu.SemaphoreType.DMA,
            pltpu.SemaphoreType.DMA,
            pltpu.SemaphoreType.DMA,
        ],
    )
    def k(table_hbm, idx_hbm, out_hbm, idx_v,
          b0, b1, b2, b3, g0, g1, g2, g3, w0, w1, w2, w3):
        wid = lax.axis_index("s") * _NC + lax.axis_index("c")
        pltpu.sync_copy(idx_hbm.at[wid], idx_v)
        base = wid * n_chunks

        def sg(j, buf, sem):
            pltpu.make_async_copy(table_hbm.at[idx_v.at[j]], buf, sem).start()

        def sw(j, buf, sem):
            off = jnp.minimum(base + j, total_chunks - 1) * _C
            pltpu.make_async_copy(buf, out_hbm.at[pl.ds(off, _C)], sem).start()

        def gwait(buf, sem):
            pltpu.make_async_copy(table_hbm.at[idx_v.at[0]], buf, sem).wait()

        def wwait(buf, sem):
            pltpu.make_async_copy(buf, out_hbm.at[pl.ds(0, _C)], sem).wait()

        sg(0, b0, g0)
        sg(1, b1, g1)

        @pl.loop(0, n_chunks // 4)
        def _(p):
            j0 = 4 * p
            gwait(b0, g0)

            @pl.when(p > 0)
            def _():
                wwait(b2, w2)

            sg(j0 + 2, b2, g2)
            sw(j0, b0, w0)

            gwait(b1, g1)

            @pl.when(p > 0)
            def _():
                wwait(b3, w3)

            sg(j0 + 3, b3, g3)
            sw(j0 + 1, b1, w1)

            gwait(b2, g2)
            wwait(b0, w0)
            sg(j0 + 4, b0, g0)
            sw(j0 + 2, b2, w2)

            gwait(b3, g3)
            wwait(b1, w1)
            sg(j0 + 5, b1, g1)
            sw(j0 + 3, b3, w3)

        gwait(b0, g0)
        wwait(b2, w2)
        sw(n_chunks - 2, b0, w0)
        gwait(b1, g1)
        wwait(b3, w3)
        sw(n_chunks - 1, b1, w1)
        wwait(b0, w0)
        wwait(b1, w1)

    return k(table, idx3d)


def kernel(tokenized_prompts, token_embedding_table):
    n, ctx = tokenized_prompts.shape
    _, d = token_embedding_table.shape
    b = n * ctx
    sweep = _C * _NW
    b_pad = ((b + sweep - 1) // sweep) * sweep
    n_chunks = b_pad // sweep
    if n_chunks % 4 != 2:
        n_chunks += (2 - n_chunks % 4) % 4
        b_pad = n_chunks * sweep
    flat = tokenized_prompts.T.reshape(-1)
    pad_chunks = (b_pad - b) // _C
    if pad_chunks:
        flat = jnp.concatenate([flat] + [flat[b - _C:]] * pad_chunks)
    idx3d = flat.reshape(_NW, n_chunks, _C)
    out = _sc_gather(token_embedding_table, idx3d, n_chunks, b, d)
    prompts = out.reshape(ctx, n, d).transpose(1, 0, 2)
    return (prompts, tokenized_prompts)


# speedup vs baseline: 1.0174x; 1.0101x over previous
import functools

import jax
import jax.numpy as jnp
from jax import lax
from jax.experimental import pallas as pl
from jax.experimental.pallas import tpu as pltpu
from jax.experimental.pallas import tpu_sc as plsc

_NC = 2
_NS = 16
_NW = _NC * _NS
_C = 112


def _sc_gather(table, idx3d, n_chunks, out_rows, d):
    mesh = plsc.VectorSubcoreMesh(core_axis_name="c", subcore_axis_name="s")
    total_chunks = out_rows // _C

    assert n_chunks % 2 == 1

    @functools.partial(
        pl.kernel,
        out_type=jax.ShapeDtypeStruct((out_rows, d), table.dtype),
        mesh=mesh,
        scratch_types=[
            pltpu.VMEM((n_chunks, _C), jnp.int32),
            pltpu.VMEM((_C, d), table.dtype),
            pltpu.VMEM((_C, d), table.dtype),
            pltpu.SemaphoreType.DMA,
            pltpu.SemaphoreType.DMA,
            pltpu.SemaphoreType.DMA,
            pltpu.SemaphoreType.DMA,
        ],
    )
    def k(table_hbm, idx_hbm, out_hbm, idx_v, rows0, rows1, g0, g1, w0, w1):
        wid = lax.axis_index("s") * _NC + lax.axis_index("c")
        pltpu.sync_copy(idx_hbm.at[wid], idx_v)
        base = wid * n_chunks

        def sg(j, buf, sem):
            pltpu.make_async_copy(table_hbm.at[idx_v.at[j]], buf, sem).start()

        def out_ref(j):
            off = jnp.minimum(base + j, total_chunks - 1) * _C
            return out_hbm.at[pl.ds(off, _C)]

        def sw(j, buf, sem):
            pltpu.make_async_copy(buf, out_ref(j), sem).start()

        def gwait(buf, sem):
            pltpu.make_async_copy(table_hbm.at[idx_v.at[0]], buf, sem).wait()

        def wwait(buf, sem):
            pltpu.make_async_copy(buf, out_hbm.at[pl.ds(0, _C)], sem).wait()

        sg(0, rows0, g0)

        @pl.loop(0, n_chunks // 2)
        def _(p):
            j0 = 2 * p
            gwait(rows0, g0)

            @pl.when(p > 0)
            def _():
                wwait(rows1, w1)

            sg(j0 + 1, rows1, g1)
            sw(j0, rows0, w0)
            gwait(rows1, g1)
            wwait(rows0, w0)
            sg(j0 + 2, rows0, g0)
            sw(j0 + 1, rows1, w1)

        gwait(rows0, g0)
        wwait(rows1, w1)
        sw(n_chunks - 1, rows0, w0)
        wwait(rows0, w0)

    return k(table, idx3d)


def kernel(tokenized_prompts, token_embedding_table):
    n, ctx = tokenized_prompts.shape
    _, d = token_embedding_table.shape
    b = n * ctx
    sweep = _C * _NW
    b_pad = ((b + sweep - 1) // sweep) * sweep
    n_chunks = b_pad // sweep
    flat = tokenized_prompts.T.reshape(-1)
    pad_chunks = (b_pad - b) // _C
    if pad_chunks:
        flat = jnp.concatenate([flat] + [flat[b - _C:]] * pad_chunks)
    idx3d = flat.reshape(_NW, n_chunks, _C)
    out = _sc_gather(token_embedding_table, idx3d, n_chunks, b, d)
    prompts = out.reshape(ctx, n, d).transpose(1, 0, 2)
    return (prompts, tokenized_prompts)
